# baseline (speedup 1.0000x reference)
import functools

import jax
import jax.numpy as jnp
from jax.experimental import pallas as pl
import jax.experimental.pallas.tpu as pltpu

N = 1024
IN_DIM = 128
OUT_DIM = 64
BR = 1024
GRID = N // BR


def _gat_body(h_ref, adj_ref, w_ref, a_ref, out_ref, wh_ref, s_ref, t_ref):
    i = pl.program_id(0)

    @pl.when(i == 0)
    def _precompute():
        wh = jnp.dot(h_ref[...], w_ref[...], preferred_element_type=jnp.float32)
        wh_ref[...] = wh
        a1 = -a_ref[..., :OUT_DIM]
        a2 = -a_ref[..., OUT_DIM:]
        s_ref[...] = jax.lax.dot_general(
            wh, a1, (((1,), (1,)), ((), ())), preferred_element_type=jnp.float32)
        t_ref[...] = jax.lax.dot_general(
            a2, wh, (((1,), (1,)), ((), ())), preferred_element_type=jnp.float32)

    s_blk = s_ref[pl.ds(i * BR, BR), :]
    x = s_blk + t_ref[...]
    e = jnp.exp(jnp.minimum(x, 0.2 * x))
    e = e * adj_ref[...].astype(jnp.float32)
    rowsum = jnp.sum(e, axis=1, keepdims=True)
    hp = jnp.dot(e, wh_ref[...], preferred_element_type=jnp.float32)
    hp = hp / rowsum
    out_ref[...] = jnp.where(hp > 0, hp, jnp.exp(jnp.minimum(hp, 0.0)) - 1.0)


@jax.jit
def kernel(h, adj, W, a):
    return pl.pallas_call(
        _gat_body,
        grid=(GRID,),
        in_specs=[
            pl.BlockSpec((N, IN_DIM), lambda i: (0, 0)),
            pl.BlockSpec((BR, N), lambda i: (i, 0)),
            pl.BlockSpec((IN_DIM, OUT_DIM), lambda i: (0, 0)),
            pl.BlockSpec((1, 2 * OUT_DIM), lambda i: (0, 0)),
        ],
        out_specs=pl.BlockSpec((BR, OUT_DIM), lambda i: (i, 0)),
        out_shape=jax.ShapeDtypeStruct((N, OUT_DIM), jnp.float32),
        scratch_shapes=[
            pltpu.VMEM((N, OUT_DIM), jnp.float32),
            pltpu.VMEM((N, 1), jnp.float32),
            pltpu.VMEM((1, N), jnp.float32),
        ],
    )(h, adj, W, a)


# speedup vs baseline: 1.0218x; 1.0218x over previous
import functools

import jax
import jax.numpy as jnp
from jax.experimental import pallas as pl
import jax.experimental.pallas.tpu as pltpu

N = 1024
IN_DIM = 128
OUT_DIM = 64
BR = 512
GRID = N // BR


def _gat_body(h_ref, adj_ref, w_ref, a_ref, out_ref, wh_ref, whb_ref, s_ref, t_ref):
    i = pl.program_id(0)

    @pl.when(i == 0)
    def _precompute():
        wh = jnp.dot(h_ref[...], w_ref[...], preferred_element_type=jnp.float32)
        wh_ref[...] = wh
        whb_ref[...] = wh.astype(jnp.bfloat16)
        a1 = -a_ref[..., :OUT_DIM]
        a2 = -a_ref[..., OUT_DIM:]
        s_ref[...] = jax.lax.dot_general(
            wh, a1, (((1,), (1,)), ((), ())), preferred_element_type=jnp.float32)
        t_ref[...] = jax.lax.dot_general(
            a2, wh, (((1,), (1,)), ((), ())), preferred_element_type=jnp.float32)

    s_blk = s_ref[pl.ds(i * BR, BR), :]
    x = s_blk + t_ref[...]
    e = jnp.exp(jnp.minimum(x, 0.2 * x))
    e = e * adj_ref[...].astype(jnp.float32)
    rowsum = jnp.sum(e, axis=1, keepdims=True)
    hp = jnp.dot(e.astype(jnp.bfloat16), whb_ref[...],
                 preferred_element_type=jnp.float32)
    hp = hp / rowsum
    out_ref[...] = jnp.where(hp > 0, hp, jnp.exp(jnp.minimum(hp, 0.0)) - 1.0)


@jax.jit
def kernel(h, adj, W, a):
    return pl.pallas_call(
        _gat_body,
        grid=(GRID,),
        in_specs=[
            pl.BlockSpec((N, IN_DIM), lambda i: (0, 0)),
            pl.BlockSpec((BR, N), lambda i: (i, 0)),
            pl.BlockSpec((IN_DIM, OUT_DIM), lambda i: (0, 0)),
            pl.BlockSpec((1, 2 * OUT_DIM), lambda i: (0, 0)),
        ],
        out_specs=pl.BlockSpec((BR, OUT_DIM), lambda i: (i, 0)),
        out_shape=jax.ShapeDtypeStruct((N, OUT_DIM), jnp.float32),
        scratch_shapes=[
            pltpu.VMEM((N, OUT_DIM), jnp.float32),
            pltpu.VMEM((N, OUT_DIM), jnp.bfloat16),
            pltpu.VMEM((N, 1), jnp.float32),
            pltpu.VMEM((1, N), jnp.float32),
        ],
    )(h, adj, W, a)


# speedup vs baseline: 1.0922x; 1.0688x over previous
import jax
import jax.numpy as jnp
from jax.experimental import pallas as pl
import jax.experimental.pallas.tpu as pltpu

N = 1024
IN_DIM = 128
OUT_DIM = 64
BR = 512
GRID = N // BR


def _gat_body(h_ref, adj_ref, w_ref, a_ref, out_ref,
              whb_ref, u_ref, p_ref, v_ref, q_ref):
    i = pl.program_id(0)

    @pl.when(i == 0)
    def _precompute():
        wh = jnp.dot(h_ref[...], w_ref[...], preferred_element_type=jnp.float32)
        whb_ref[:, :OUT_DIM] = wh.astype(jnp.bfloat16)
        whb_ref[:, OUT_DIM:] = jnp.full((N, OUT_DIM), 0, jnp.bfloat16)
        whb_ref[:, OUT_DIM:OUT_DIM + 1] = jnp.full((N, 1), 1, jnp.bfloat16)
        a1 = -a_ref[..., :OUT_DIM]
        a2 = -a_ref[..., OUT_DIM:]
        s = jax.lax.dot_general(
            wh, a1, (((1,), (1,)), ((), ())), preferred_element_type=jnp.float32)
        t = jax.lax.dot_general(
            a2, wh, (((1,), (1,)), ((), ())), preferred_element_type=jnp.float32)
        u_ref[...] = jnp.exp(s)
        p_ref[...] = jnp.exp(0.2 * s)
        v_ref[...] = jnp.exp(t)
        q_ref[...] = jnp.exp(0.2 * t)

    u = u_ref[pl.ds(i * BR, BR), :]
    p = p_ref[pl.ds(i * BR, BR), :]
    e = jnp.minimum(u * v_ref[...], p * q_ref[...])
    e = e * adj_ref[...].astype(jnp.float32)
    hpf = jnp.dot(e.astype(jnp.bfloat16), whb_ref[...],
                  preferred_element_type=jnp.float32)
    hp = hpf[:, :OUT_DIM] / hpf[:, OUT_DIM:OUT_DIM + 1]
    out_ref[...] = jnp.where(hp > 0, hp, jnp.exp(jnp.minimum(hp, 0.0)) - 1.0)


@jax.jit
def kernel(h, adj, W, a):
    return pl.pallas_call(
        _gat_body,
        grid=(GRID,),
        in_specs=[
            pl.BlockSpec((N, IN_DIM), lambda i: (0, 0)),
            pl.BlockSpec((BR, N), lambda i: (i, 0)),
            pl.BlockSpec((IN_DIM, OUT_DIM), lambda i: (0, 0)),
            pl.BlockSpec((1, 2 * OUT_DIM), lambda i: (0, 0)),
        ],
        out_specs=pl.BlockSpec((BR, OUT_DIM), lambda i: (i, 0)),
        out_shape=jax.ShapeDtypeStruct((N, OUT_DIM), jnp.float32),
        scratch_shapes=[
            pltpu.VMEM((N, 2 * OUT_DIM), jnp.bfloat16),
            pltpu.VMEM((N, 1), jnp.float32),
            pltpu.VMEM((N, 1), jnp.float32),
            pltpu.VMEM((1, N), jnp.float32),
            pltpu.VMEM((1, N), jnp.float32),
        ],
    )(h, adj, W, a)


# speedup vs baseline: 1.0999x; 1.0071x over previous
import jax
import jax.numpy as jnp
from jax.experimental import pallas as pl
import jax.experimental.pallas.tpu as pltpu

N = 1024
IN_DIM = 128
OUT_DIM = 64
BR = 512
GRID = N // BR


def _gat_body(h_ref, adj_ref, w_ref, a_ref, out_ref,
              whb_ref, u_ref, p_ref, v_ref, q_ref):
    i = pl.program_id(0)

    @pl.when(i == 0)
    def _precompute():
        wh = jnp.dot(h_ref[...], w_ref[...], preferred_element_type=jnp.float32)
        whb_ref[:, :OUT_DIM] = wh.astype(jnp.bfloat16)
        whb_ref[:, OUT_DIM:] = jnp.full((N, OUT_DIM), 0, jnp.bfloat16)
        whb_ref[:, OUT_DIM:OUT_DIM + 1] = jnp.full((N, 1), 1, jnp.bfloat16)
        a1 = -a_ref[..., :OUT_DIM]
        a2 = -a_ref[..., OUT_DIM:]
        s = jax.lax.dot_general(
            wh, a1, (((1,), (1,)), ((), ())), preferred_element_type=jnp.float32)
        t = jax.lax.dot_general(
            a2, wh, (((1,), (1,)), ((), ())), preferred_element_type=jnp.float32)
        u_ref[...] = jnp.exp(s).astype(jnp.bfloat16)
        p_ref[...] = jnp.exp(0.2 * s).astype(jnp.bfloat16)
        v_ref[...] = jnp.exp(t).astype(jnp.bfloat16)
        q_ref[...] = jnp.exp(0.2 * t).astype(jnp.bfloat16)

    u = u_ref[pl.ds(i * BR, BR), :]
    p = p_ref[pl.ds(i * BR, BR), :]
    e = jnp.minimum(u * v_ref[...], p * q_ref[...])
    e = e * adj_ref[...].astype(jnp.bfloat16)
    hpf = jnp.dot(e, whb_ref[...],
                  preferred_element_type=jnp.float32)
    hp = hpf[:, :OUT_DIM] / hpf[:, OUT_DIM:OUT_DIM + 1]
    out_ref[...] = jnp.where(hp > 0, hp, jnp.exp(jnp.minimum(hp, 0.0)) - 1.0)


@jax.jit
def kernel(h, adj, W, a):
    return pl.pallas_call(
        _gat_body,
        grid=(GRID,),
        in_specs=[
            pl.BlockSpec((N, IN_DIM), lambda i: (0, 0)),
            pl.BlockSpec((BR, N), lambda i: (i, 0)),
            pl.BlockSpec((IN_DIM, OUT_DIM), lambda i: (0, 0)),
            pl.BlockSpec((1, 2 * OUT_DIM), lambda i: (0, 0)),
        ],
        out_specs=pl.BlockSpec((BR, OUT_DIM), lambda i: (i, 0)),
        out_shape=jax.ShapeDtypeStruct((N, OUT_DIM), jnp.float32),
        scratch_shapes=[
            pltpu.VMEM((N, 2 * OUT_DIM), jnp.bfloat16),
            pltpu.VMEM((N, 1), jnp.bfloat16),
            pltpu.VMEM((N, 1), jnp.bfloat16),
            pltpu.VMEM((1, N), jnp.bfloat16),
            pltpu.VMEM((1, N), jnp.bfloat16),
        ],
    )(h, adj, W, a)
